# baseline (speedup 1.0000x reference)
import jax
import jax.numpy as jnp
from jax.experimental import pallas as pl
from jax.experimental.pallas import tpu as pltpu

K = 2
B_T = 256
F_TILE = 1024


def _ffn_kernel(be_ref, x_ref, w1_ref, w3_ref, w2_ref, out_ref, acc_ref):
    f = pl.program_id(1)
    j = pl.program_id(2)
    nf = pl.num_programs(1)
    x = x_ref[...]
    h1 = jnp.dot(x, w1_ref[0], preferred_element_type=jnp.float32)
    h3 = jnp.dot(x, w3_ref[0], preferred_element_type=jnp.float32)
    act = h1 * jax.nn.sigmoid(h1) * h3
    part = jnp.dot(act, w2_ref[0], preferred_element_type=jnp.float32)
    row = pl.multiple_of(j * B_T, B_T)

    @pl.when(f == 0)
    def _():
        acc_ref[pl.ds(row, B_T), :] = part

    @pl.when((f != 0) & (f != nf - 1))
    def _():
        acc_ref[pl.ds(row, B_T), :] += part

    @pl.when(f == nf - 1)
    def _():
        out_ref[...] = acc_ref[pl.ds(row, B_T), :] + part


def kernel(x, Wr, W1, W2, W3):
    b, s, h = x.shape
    T = b * s
    E = Wr.shape[1]
    F = W1.shape[2]
    A = T * K
    NF = F // F_TILE
    NB = A // B_T + E
    P = NB * B_T

    xf = x.reshape(T, h)

    logits = xf @ Wr
    top_vals, top_idx = jax.lax.top_k(logits, K)
    rw = jax.nn.softmax(top_vals, axis=-1)

    ef = top_idx.reshape(A)
    onehot = (ef[:, None] == jnp.arange(E, dtype=ef.dtype)[None, :]).astype(jnp.int32)
    counts = onehot.sum(0)
    csum = jnp.cumsum(counts)
    offsets = csum - counts
    rank = jnp.take_along_axis(jnp.cumsum(onehot, axis=0), ef[:, None], axis=1)[:, 0] - 1

    nblk = (counts + B_T - 1) // B_T
    blk_incl = jnp.cumsum(nblk)
    blk_excl = blk_incl - nblk
    block_e = jnp.clip(
        jnp.searchsorted(blk_incl, jnp.arange(NB, dtype=jnp.int32), side="right"),
        0, E - 1).astype(jnp.int32)

    p_a = (blk_excl[ef] + rank // B_T) * B_T + rank % B_T

    tok = jnp.arange(A, dtype=jnp.int32) // K
    tok_pad = jnp.zeros((P,), jnp.int32).at[p_a].set(tok)
    xs_pad = xf[tok_pad]

    NC = 2
    NB2 = NB // NC
    grid_spec = pltpu.PrefetchScalarGridSpec(
        num_scalar_prefetch=1,
        grid=(NC, NF, NB2),
        in_specs=[
            pl.BlockSpec((B_T, h), lambda g, f, j, be: (g * NB2 + j, 0)),
            pl.BlockSpec((1, h, F_TILE), lambda g, f, j, be: (be[g * NB2 + j], 0, f)),
            pl.BlockSpec((1, h, F_TILE), lambda g, f, j, be: (be[g * NB2 + j], 0, f)),
            pl.BlockSpec((1, F_TILE, h), lambda g, f, j, be: (be[g * NB2 + j], f, 0)),
        ],
        out_specs=pl.BlockSpec(
            (B_T, h),
            lambda g, f, j, be: (g * NB2 + jnp.where(f == NF - 1, j, 0), 0)),
        scratch_shapes=[pltpu.VMEM((NB2 * B_T, h), jnp.float32)],
    )
    Y = pl.pallas_call(
        _ffn_kernel,
        grid_spec=grid_spec,
        out_shape=jax.ShapeDtypeStruct((P, h), jnp.float32),
        compiler_params=pltpu.CompilerParams(
            dimension_semantics=("parallel", "arbitrary", "arbitrary"),
            vmem_limit_bytes=56 * 1024 * 1024,
        ),
    )(block_e, xs_pad, W1, W3, W2)

    sel = Y[p_a].reshape(T, K, h)
    final = (sel * rw[:, :, None]).sum(1).reshape(b, s, h)
    metrics = jnp.sqrt((sel * sel).sum(-1)).reshape(b, s, K)
    return (final,
            rw.reshape(b, s, K),
            top_idx.reshape(b, s, K),
            metrics)


# speedup vs baseline: 1.0244x; 1.0244x over previous
import jax
import jax.numpy as jnp
from jax.experimental import pallas as pl
from jax.experimental.pallas import tpu as pltpu

K = 2
B_T = 256
F_TILE = 1024


def _ffn_kernel(be_ref, x_ref, w1_ref, w3_ref, w2_ref, out_ref, outn_ref, acc_ref):
    f = pl.program_id(0)
    j = pl.program_id(1)
    nf = pl.num_programs(0)
    x = x_ref[...]
    h1 = jnp.dot(x, w1_ref[0], preferred_element_type=jnp.float32)
    h3 = jnp.dot(x, w3_ref[0], preferred_element_type=jnp.float32)
    act = h1 * jax.nn.sigmoid(h1) * h3
    part = jnp.dot(act, w2_ref[0], preferred_element_type=jnp.float32)
    row = pl.multiple_of(j * B_T, B_T)

    @pl.when(f == 0)
    def _():
        acc_ref[pl.ds(row, B_T), :] = part

    @pl.when((f != 0) & (f != nf - 1))
    def _():
        acc_ref[pl.ds(row, B_T), :] += part

    @pl.when(f == nf - 1)
    def _():
        total = acc_ref[pl.ds(row, B_T), :] + part
        out_ref[...] = total
        outn_ref[...] = jnp.sum(total * total, axis=1, keepdims=True)


def kernel(x, Wr, W1, W2, W3):
    b, s, h = x.shape
    T = b * s
    E = Wr.shape[1]
    F = W1.shape[2]
    A = T * K
    NF = F // F_TILE
    NB = A // B_T + E
    P = NB * B_T

    xf = x.reshape(T, h)

    logits = xf @ Wr
    top_vals, top_idx = jax.lax.top_k(logits, K)
    rw = jax.nn.softmax(top_vals, axis=-1)

    ef = top_idx.reshape(A)
    onehot = (ef[:, None] == jnp.arange(E, dtype=ef.dtype)[None, :]).astype(jnp.int32)
    counts = onehot.sum(0)
    csum = jnp.cumsum(counts)
    offsets = csum - counts
    rank = jnp.take_along_axis(jnp.cumsum(onehot, axis=0), ef[:, None], axis=1)[:, 0] - 1

    nblk = (counts + B_T - 1) // B_T
    blk_incl = jnp.cumsum(nblk)
    blk_excl = blk_incl - nblk
    block_e = jnp.clip(
        jnp.searchsorted(blk_incl, jnp.arange(NB, dtype=jnp.int32), side="right"),
        0, E - 1).astype(jnp.int32)

    p_a = (blk_excl[ef] + rank // B_T) * B_T + rank % B_T

    tok = jnp.arange(A, dtype=jnp.int32) // K
    tok_pad = jnp.zeros((P,), jnp.int32).at[p_a].set(tok)
    xs_pad = xf.astype(jnp.bfloat16)[tok_pad]

    grid_spec = pltpu.PrefetchScalarGridSpec(
        num_scalar_prefetch=1,
        grid=(NF, NB),
        in_specs=[
            pl.BlockSpec((B_T, h), lambda f, j, be: (j, 0)),
            pl.BlockSpec((1, h, F_TILE), lambda f, j, be: (be[j], 0, f)),
            pl.BlockSpec((1, h, F_TILE), lambda f, j, be: (be[j], 0, f)),
            pl.BlockSpec((1, F_TILE, h), lambda f, j, be: (be[j], f, 0)),
        ],
        out_specs=[
            pl.BlockSpec(
                (B_T, h), lambda f, j, be: (jnp.where(f == NF - 1, j, 0), 0)),
            pl.BlockSpec(
                (B_T, 1), lambda f, j, be: (jnp.where(f == NF - 1, j, 0), 0)),
        ],
        scratch_shapes=[pltpu.VMEM((P, h), jnp.float32)],
    )
    Y, Yn = pl.pallas_call(
        _ffn_kernel,
        grid_spec=grid_spec,
        out_shape=[jax.ShapeDtypeStruct((P, h), jnp.float32),
                   jax.ShapeDtypeStruct((P, 1), jnp.float32)],
        compiler_params=pltpu.CompilerParams(
            dimension_semantics=("arbitrary", "arbitrary"),
            vmem_limit_bytes=56 * 1024 * 1024,
        ),
    )(block_e, xs_pad, W1, W3, W2)

    sel = Y[p_a].reshape(T, K, h)
    final = (sel * rw[:, :, None]).sum(1).reshape(b, s, h)
    metrics = jnp.sqrt(Yn[:, 0][p_a]).reshape(b, s, K)
    return (final,
            rw.reshape(b, s, K),
            top_idx.reshape(b, s, K),
            metrics)


# speedup vs baseline: 1.0595x; 1.0343x over previous
import functools

import jax
import jax.numpy as jnp
from jax import lax
from jax.experimental import pallas as pl
from jax.experimental.pallas import tpu as pltpu
from jax.experimental.pallas import tpu_sc as plsc

K = 2
B_T = 256
F_TILE = 1024


def _ffn_kernel(be_ref, x_ref, w1_ref, w3_ref, w2_ref, wt_ref,
                out_ref, outn_ref, acc_ref):
    f = pl.program_id(0)
    j = pl.program_id(1)
    nf = pl.num_programs(0)
    x = x_ref[...]
    h1 = jnp.dot(x, w1_ref[0], preferred_element_type=jnp.float32)
    h3 = jnp.dot(x, w3_ref[0], preferred_element_type=jnp.float32)
    act = h1 * jax.nn.sigmoid(h1) * h3
    part = jnp.dot(act, w2_ref[0], preferred_element_type=jnp.float32)
    row = pl.multiple_of(j * B_T, B_T)

    @pl.when(f == 0)
    def _():
        acc_ref[pl.ds(row, B_T), :] = part

    @pl.when((f != 0) & (f != nf - 1))
    def _():
        acc_ref[pl.ds(row, B_T), :] += part

    @pl.when(f == nf - 1)
    def _():
        total = acc_ref[pl.ds(row, B_T), :] + part
        outn_ref[...] = jnp.sum(total * total, axis=1, keepdims=True)
        out_ref[...] = total * wt_ref[...]


def _sc_combine(yw, pa, T, h):
    NC, NS = 2, 16
    NW = NC * NS
    TPW = T // NW
    CT = 16
    NCH = TPW // CT

    mesh = plsc.VectorSubcoreMesh(core_axis_name="c", subcore_axis_name="s")

    @functools.partial(
        pl.kernel,
        out_type=jax.ShapeDtypeStruct((T, h), jnp.float32),
        mesh=mesh,
        scratch_types=[
            pltpu.VMEM((2 * TPW,), jnp.int32),
            pltpu.VMEM((2 * CT, h), jnp.float32),
            pltpu.VMEM((CT, h), jnp.float32),
            pltpu.SemaphoreType.DMA,
        ],
    )
    def combine(yw_hbm, pa_hbm, out_hbm, idx_v, rows_v, out_v, sem):
        wid = lax.axis_index("s") * NC + lax.axis_index("c")
        abase = wid * (2 * TPW)
        pltpu.sync_copy(pa_hbm.at[pl.ds(abase, 2 * TPW)], idx_v)
        for c in range(NCH):
            pltpu.async_copy(
                yw_hbm.at[idx_v.at[pl.ds(c * 2 * CT, 2 * CT)]], rows_v, sem
            ).wait()

            def body(i, carry):
                for q in range(h // 16):
                    sl = pl.ds(q * 16, 16)
                    out_v[i, sl] = rows_v[2 * i, sl] + rows_v[2 * i + 1, sl]
                return carry

            lax.fori_loop(0, CT, body, 0)
            pltpu.sync_copy(out_v, out_hbm.at[pl.ds(wid * TPW + c * CT, CT)])

    return combine(yw, pa)


def kernel(x, Wr, W1, W2, W3):
    b, s, h = x.shape
    T = b * s
    E = Wr.shape[1]
    F = W1.shape[2]
    A = T * K
    NF = F // F_TILE
    NB = A // B_T + E
    P = NB * B_T

    xf = x.reshape(T, h)

    logits = xf @ Wr
    top_vals, top_idx = jax.lax.top_k(logits, K)
    rw = jax.nn.softmax(top_vals, axis=-1)

    ef = top_idx.reshape(A)
    onehot = (ef[:, None] == jnp.arange(E, dtype=ef.dtype)[None, :]).astype(jnp.int32)
    counts = onehot.sum(0)
    csum = jnp.cumsum(counts)
    offsets = csum - counts
    rank = jnp.take_along_axis(jnp.cumsum(onehot, axis=0), ef[:, None], axis=1)[:, 0] - 1

    nblk = (counts + B_T - 1) // B_T
    blk_incl = jnp.cumsum(nblk)
    blk_excl = blk_incl - nblk
    block_e = jnp.clip(
        jnp.searchsorted(blk_incl, jnp.arange(NB, dtype=jnp.int32), side="right"),
        0, E - 1).astype(jnp.int32)

    p_a = (blk_excl[ef] + rank // B_T) * B_T + rank % B_T

    tok = jnp.arange(A, dtype=jnp.int32) // K
    tw = jnp.stack([tok.astype(jnp.float32), rw.reshape(A)], axis=1)
    tw_pad = jnp.zeros((P, 2), jnp.float32).at[p_a].set(tw)
    tok_pad = tw_pad[:, 0].astype(jnp.int32)
    w_pad = tw_pad[:, 1:2]
    xs_pad = xf.astype(jnp.bfloat16)[tok_pad]

    grid_spec = pltpu.PrefetchScalarGridSpec(
        num_scalar_prefetch=1,
        grid=(NF, NB),
        in_specs=[
            pl.BlockSpec((B_T, h), lambda f, j, be: (j, 0)),
            pl.BlockSpec((1, h, F_TILE), lambda f, j, be: (be[j], 0, f)),
            pl.BlockSpec((1, h, F_TILE), lambda f, j, be: (be[j], 0, f)),
            pl.BlockSpec((1, F_TILE, h), lambda f, j, be: (be[j], f, 0)),
            pl.BlockSpec((B_T, 1), lambda f, j, be: (j, 0)),
        ],
        out_specs=[
            pl.BlockSpec(
                (B_T, h), lambda f, j, be: (jnp.where(f == NF - 1, j, 0), 0)),
            pl.BlockSpec(
                (B_T, 1), lambda f, j, be: (jnp.where(f == NF - 1, j, 0), 0)),
        ],
        scratch_shapes=[pltpu.VMEM((P, h), jnp.float32)],
    )
    Y, Yn = pl.pallas_call(
        _ffn_kernel,
        grid_spec=grid_spec,
        out_shape=[jax.ShapeDtypeStruct((P, h), jnp.float32),
                   jax.ShapeDtypeStruct((P, 1), jnp.float32)],
        compiler_params=pltpu.CompilerParams(
            dimension_semantics=("arbitrary", "arbitrary"),
            vmem_limit_bytes=56 * 1024 * 1024,
        ),
    )(block_e, xs_pad, W1, W3, W2, w_pad)

    final = _sc_combine(Y, p_a, T, h).reshape(b, s, h)
    metrics = jnp.sqrt(Yn[:, 0][p_a]).reshape(b, s, K)
    return (final,
            rw.reshape(b, s, K),
            top_idx.reshape(b, s, K),
            metrics)


# speedup vs baseline: 1.0740x; 1.0136x over previous
import functools

import jax
import jax.numpy as jnp
from jax import lax
from jax.experimental import pallas as pl
from jax.experimental.pallas import tpu as pltpu
from jax.experimental.pallas import tpu_sc as plsc

K = 2
B_T = 512
F_TILE = 1024


def _ffn_kernel(be_ref, x_ref, w1_ref, w3_ref, w2_ref, wt_ref,
                out_ref, outn_ref, acc_ref):
    f = pl.program_id(0)
    j = pl.program_id(1)
    nf = pl.num_programs(0)
    x = x_ref[...]
    h1 = jnp.dot(x, w1_ref[0], preferred_element_type=jnp.float32)
    h3 = jnp.dot(x, w3_ref[0], preferred_element_type=jnp.float32)
    act = h1 * jax.nn.sigmoid(h1) * h3
    part = jnp.dot(act, w2_ref[0], preferred_element_type=jnp.float32)
    row = pl.multiple_of(j * B_T, B_T)

    @pl.when(f == 0)
    def _():
        acc_ref[pl.ds(row, B_T), :] = part.astype(acc_ref.dtype)

    @pl.when((f != 0) & (f != nf - 1))
    def _():
        acc_ref[pl.ds(row, B_T), :] = (
            acc_ref[pl.ds(row, B_T), :] + part).astype(acc_ref.dtype)

    @pl.when(f == nf - 1)
    def _():
        total = acc_ref[pl.ds(row, B_T), :] + part
        outn_ref[...] = jnp.sum(total * total, axis=1, keepdims=True)
        out_ref[...] = total * wt_ref[...]


def _sc_combine(yw, pa, T, h):
    NC, NS = 2, 16
    NW = NC * NS
    TPW = T // NW
    CT = 16
    NCH = TPW // CT

    mesh = plsc.VectorSubcoreMesh(core_axis_name="c", subcore_axis_name="s")

    @functools.partial(
        pl.kernel,
        out_type=jax.ShapeDtypeStruct((T, h), jnp.float32),
        mesh=mesh,
        scratch_types=[
            pltpu.VMEM((2 * TPW,), jnp.int32),
            pltpu.VMEM((2 * CT, h), jnp.float32),
            pltpu.VMEM((CT, h), jnp.float32),
            pltpu.SemaphoreType.DMA,
        ],
    )
    def combine(yw_hbm, pa_hbm, out_hbm, idx_v, rows_v, out_v, sem):
        wid = lax.axis_index("s") * NC + lax.axis_index("c")
        abase = wid * (2 * TPW)
        pltpu.sync_copy(pa_hbm.at[pl.ds(abase, 2 * TPW)], idx_v)
        for c in range(NCH):
            pltpu.async_copy(
                yw_hbm.at[idx_v.at[pl.ds(c * 2 * CT, 2 * CT)]], rows_v, sem
            ).wait()

            def body(i, carry):
                for q in range(h // 16):
                    sl = pl.ds(q * 16, 16)
                    out_v[i, sl] = rows_v[2 * i, sl] + rows_v[2 * i + 1, sl]
                return carry

            lax.fori_loop(0, CT, body, 0)
            pltpu.sync_copy(out_v, out_hbm.at[pl.ds(wid * TPW + c * CT, CT)])

    return combine(yw, pa)


def kernel(x, Wr, W1, W2, W3):
    b, s, h = x.shape
    T = b * s
    E = Wr.shape[1]
    F = W1.shape[2]
    A = T * K
    NF = F // F_TILE
    NB = A // B_T + E
    P = NB * B_T

    xf = x.reshape(T, h)

    logits = xf @ Wr
    top_vals, top_idx = jax.lax.top_k(logits, K)
    rw = jax.nn.softmax(top_vals, axis=-1)

    ef = top_idx.reshape(A)
    onehot = (ef[:, None] == jnp.arange(E, dtype=ef.dtype)[None, :]).astype(jnp.int32)
    counts = onehot.sum(0)
    csum = jnp.cumsum(counts)
    offsets = csum - counts
    rank = jnp.take_along_axis(jnp.cumsum(onehot, axis=0), ef[:, None], axis=1)[:, 0] - 1

    nblk = (counts + B_T - 1) // B_T
    blk_incl = jnp.cumsum(nblk)
    blk_excl = blk_incl - nblk
    block_e = jnp.clip(
        jnp.searchsorted(blk_incl, jnp.arange(NB, dtype=jnp.int32), side="right"),
        0, E - 1).astype(jnp.int32)

    p_a = (blk_excl[ef] + rank // B_T) * B_T + rank % B_T

    tok = jnp.arange(A, dtype=jnp.int32) // K
    tw = jnp.stack([tok.astype(jnp.float32), rw.reshape(A)], axis=1)
    tw_pad = jnp.zeros((P, 2), jnp.float32).at[p_a].set(tw)
    tok_pad = tw_pad[:, 0].astype(jnp.int32)
    w_pad = tw_pad[:, 1:2]
    xs_pad = xf.astype(jnp.bfloat16)[tok_pad]

    grid_spec = pltpu.PrefetchScalarGridSpec(
        num_scalar_prefetch=1,
        grid=(NF, NB),
        in_specs=[
            pl.BlockSpec((B_T, h), lambda f, j, be: (j, 0)),
            pl.BlockSpec((1, h, F_TILE), lambda f, j, be: (be[j], 0, f)),
            pl.BlockSpec((1, h, F_TILE), lambda f, j, be: (be[j], 0, f)),
            pl.BlockSpec((1, F_TILE, h), lambda f, j, be: (be[j], f, 0)),
            pl.BlockSpec((B_T, 1), lambda f, j, be: (j, 0)),
        ],
        out_specs=[
            pl.BlockSpec(
                (B_T, h), lambda f, j, be: (jnp.where(f == NF - 1, j, 0), 0)),
            pl.BlockSpec(
                (B_T, 1), lambda f, j, be: (jnp.where(f == NF - 1, j, 0), 0)),
        ],
        scratch_shapes=[pltpu.VMEM((P, h), jnp.bfloat16)],
    )
    Y, Yn = pl.pallas_call(
        _ffn_kernel,
        grid_spec=grid_spec,
        out_shape=[jax.ShapeDtypeStruct((P, h), jnp.float32),
                   jax.ShapeDtypeStruct((P, 1), jnp.float32)],
        compiler_params=pltpu.CompilerParams(
            dimension_semantics=("arbitrary", "arbitrary"),
            vmem_limit_bytes=56 * 1024 * 1024,
        ),
    )(block_e, xs_pad, W1, W3, W2, w_pad)

    final = _sc_combine(Y, p_a, T, h).reshape(b, s, h)
    metrics = jnp.sqrt(Yn[:, 0][p_a]).reshape(b, s, K)
    return (final,
            rw.reshape(b, s, K),
            top_idx.reshape(b, s, K),
            metrics)


# speedup vs baseline: 1.0822x; 1.0076x over previous
import functools

import jax
import jax.numpy as jnp
from jax import lax
from jax.experimental import pallas as pl
from jax.experimental.pallas import tpu as pltpu
from jax.experimental.pallas import tpu_sc as plsc

K = 2
B_T = 256
F_TILE = 1024


def _ffn_kernel(be_ref, x_ref, w1_ref, w3_ref, w2_ref, wt_ref,
                out_ref, outn_ref, acc_ref, xres_ref):
    f = pl.program_id(0)
    j = pl.program_id(1)
    nf = pl.num_programs(0)
    row = pl.multiple_of(j * B_T, B_T)

    @pl.when(f == 0)
    def _():
        xres_ref[pl.ds(row, B_T), :] = x_ref[...]

    x = xres_ref[pl.ds(row, B_T), :]
    h1 = jnp.dot(x, w1_ref[0], preferred_element_type=jnp.float32)
    h3 = jnp.dot(x, w3_ref[0], preferred_element_type=jnp.float32)
    act = h1 * jax.nn.sigmoid(h1) * h3
    part = jnp.dot(act, w2_ref[0], preferred_element_type=jnp.float32)

    @pl.when(f == 0)
    def _():
        acc_ref[pl.ds(row, B_T), :] = part.astype(acc_ref.dtype)

    @pl.when((f != 0) & (f != nf - 1))
    def _():
        acc_ref[pl.ds(row, B_T), :] = (
            acc_ref[pl.ds(row, B_T), :] + part).astype(acc_ref.dtype)

    @pl.when(f == nf - 1)
    def _():
        total = acc_ref[pl.ds(row, B_T), :] + part
        outn_ref[...] = jnp.sum(total * total, axis=1, keepdims=True)
        out_ref[...] = total * wt_ref[...]


def _sc_combine(yw, pa, T, h):
    NC, NS = 2, 16
    NW = NC * NS
    TPW = T // NW
    CT = 16
    NCH = TPW // CT

    mesh = plsc.VectorSubcoreMesh(core_axis_name="c", subcore_axis_name="s")

    @functools.partial(
        pl.kernel,
        out_type=jax.ShapeDtypeStruct((T, h), jnp.float32),
        mesh=mesh,
        scratch_types=[
            pltpu.VMEM((2 * TPW,), jnp.int32),
            pltpu.VMEM((2 * CT, h), jnp.float32),
            pltpu.VMEM((2 * CT, h), jnp.float32),
            pltpu.VMEM((CT, h), jnp.float32),
            pltpu.SemaphoreType.DMA,
            pltpu.SemaphoreType.DMA,
        ],
    )
    def combine(yw_hbm, pa_hbm, out_hbm, idx_v, rows_a, rows_b, out_v,
                sem_a, sem_b):
        wid = lax.axis_index("s") * NC + lax.axis_index("c")
        abase = wid * (2 * TPW)
        pltpu.sync_copy(pa_hbm.at[pl.ds(abase, 2 * TPW)], idx_v)
        bufs = (rows_a, rows_b)
        sems = (sem_a, sem_b)

        def gather(c):
            return pltpu.make_async_copy(
                yw_hbm.at[idx_v.at[pl.ds(c * 2 * CT, 2 * CT)]],
                bufs[c % 2], sems[c % 2])

        gather(0).start()
        for c in range(NCH):
            gather(c).wait()
            if c + 1 < NCH:
                gather(c + 1).start()
            rows_v = bufs[c % 2]

            def body(i, carry):
                for q in range(h // 16):
                    sl = pl.ds(q * 16, 16)
                    out_v[i, sl] = rows_v[2 * i, sl] + rows_v[2 * i + 1, sl]
                return carry

            lax.fori_loop(0, CT, body, 0)
            pltpu.sync_copy(out_v, out_hbm.at[pl.ds(wid * TPW + c * CT, CT)])

    return combine(yw, pa)


def kernel(x, Wr, W1, W2, W3):
    b, s, h = x.shape
    T = b * s
    E = Wr.shape[1]
    F = W1.shape[2]
    A = T * K
    NF = F // F_TILE
    NB = A // B_T + E
    P = NB * B_T

    xf = x.reshape(T, h)

    logits = xf @ Wr
    top_vals, top_idx = jax.lax.top_k(logits, K)
    rw = jax.nn.softmax(top_vals, axis=-1)

    ef = top_idx.reshape(A)
    onehot = (ef[:, None] == jnp.arange(E, dtype=ef.dtype)[None, :]).astype(jnp.int32)
    counts = onehot.sum(0)
    csum = jnp.cumsum(counts)
    offsets = csum - counts
    rank = jnp.take_along_axis(jnp.cumsum(onehot, axis=0), ef[:, None], axis=1)[:, 0] - 1

    nblk = (counts + B_T - 1) // B_T
    blk_incl = jnp.cumsum(nblk)
    blk_excl = blk_incl - nblk
    block_e = jnp.clip(
        jnp.searchsorted(blk_incl, jnp.arange(NB, dtype=jnp.int32), side="right"),
        0, E - 1).astype(jnp.int32)

    p_a = (blk_excl[ef] + rank // B_T) * B_T + rank % B_T

    tok = jnp.arange(A, dtype=jnp.int32) // K
    tw = jnp.stack([tok.astype(jnp.float32), rw.reshape(A)], axis=1)
    tw_pad = jnp.zeros((P, 2), jnp.float32).at[p_a].set(tw)
    tok_pad = tw_pad[:, 0].astype(jnp.int32)
    w_pad = tw_pad[:, 1:2]
    xs_pad = xf.astype(jnp.bfloat16)[tok_pad]

    grid_spec = pltpu.PrefetchScalarGridSpec(
        num_scalar_prefetch=1,
        grid=(NF, NB),
        in_specs=[
            pl.BlockSpec((B_T, h), lambda f, j, be: (jnp.where(f == 0, j, 0), 0)),
            pl.BlockSpec((1, h, F_TILE), lambda f, j, be: (be[j], 0, f)),
            pl.BlockSpec((1, h, F_TILE), lambda f, j, be: (be[j], 0, f)),
            pl.BlockSpec((1, F_TILE, h), lambda f, j, be: (be[j], f, 0)),
            pl.BlockSpec((B_T, 1), lambda f, j, be: (j, 0)),
        ],
        out_specs=[
            pl.BlockSpec(
                (B_T, h), lambda f, j, be: (jnp.where(f == NF - 1, j, 0), 0)),
            pl.BlockSpec(
                (B_T, 1), lambda f, j, be: (jnp.where(f == NF - 1, j, 0), 0)),
        ],
        scratch_shapes=[pltpu.VMEM((P, h), jnp.bfloat16),
                        pltpu.VMEM((P, h), jnp.bfloat16)],
    )
    Y, Yn = pl.pallas_call(
        _ffn_kernel,
        grid_spec=grid_spec,
        out_shape=[jax.ShapeDtypeStruct((P, h), jnp.float32),
                   jax.ShapeDtypeStruct((P, 1), jnp.float32)],
        compiler_params=pltpu.CompilerParams(
            dimension_semantics=("arbitrary", "arbitrary"),
            vmem_limit_bytes=56 * 1024 * 1024,
        ),
    )(block_e, xs_pad, W1, W3, W2, w_pad)

    final = _sc_combine(Y, p_a, T, h).reshape(b, s, h)
    metrics = jnp.sqrt(Yn[:, 0][p_a]).reshape(b, s, K)
    return (final,
            rw.reshape(b, s, K),
            top_idx.reshape(b, s, K),
            metrics)
